# baseline (speedup 1.0000x reference)
import dataclasses
import functools

import jax
import jax.numpy as jnp
from jax.experimental import pallas as pl
from jax.experimental.pallas import tpu as pltpu
from jax.experimental.pallas import tpu_sc as plsc

_LANES = 16
_BLOCK_ROWS = 4096
_UNROLL = 4
_TABLE_PAD = 1024
_SLABS = 2


def _sc_slab(xs, ns, t, ta, tb, mesh, cp):
    n = t.shape[0]
    c = len(xs)

    @functools.partial(
        pl.kernel,
        out_type=jax.ShapeDtypeStruct((4 * n // 128, 128), jnp.float32),
        mesh=mesh,
        compiler_params=cp,
        scratch_types=[
            pltpu.VMEM((_TABLE_PAD,), jnp.float32),
            pltpu.VMEM((_TABLE_PAD,), jnp.float32),
        ],
    )
    def _run(x0, x1, x2, n0, n1, n2, t_hbm, ta_hbm, tb_hbm, o2d, ta_v, tb_v):
        pltpu.sync_copy(ta_hbm, ta_v)
        pltpu.sync_copy(tb_hbm, tb_v)

        def body(t_v, x0v, x1v, x2v, n0v, n1v, n2v, o2v):
            @plsc.parallel_loop(0, _BLOCK_ROWS, step=_LANES, unroll=_UNROLL)
            def _(k):
                sl = pl.ds(k, _LANES)
                mm = k // 128
                r = k - mm * 128
                tv = t_v[sl]
                a = plsc.load_gather(ta_v, [tv])
                b = plsc.load_gather(tb_v, [tv])
                o2v[4 * mm + 0, pl.ds(r, _LANES)] = a * x0v[sl] + b * n0v[sl]
                o2v[4 * mm + 1, pl.ds(r, _LANES)] = a * x1v[sl] + b * n1v[sl]
                o2v[4 * mm + 2, pl.ds(r, _LANES)] = a * x2v[sl] + b * n2v[sl]

        bs = pl.BlockSpec((_BLOCK_ROWS,), lambda i: (i,))
        bso = pl.BlockSpec((4 * _BLOCK_ROWS // 128, 128), lambda i: (i, 0))
        pltpu.emit_pipeline(
            body,
            grid=(n // _BLOCK_ROWS,),
            in_specs=[bs] * (2 * c + 1),
            out_specs=[bso],
            core_axis_name=("c", "s"),
            dimension_semantics=(pltpu.PARALLEL,),
        )(t_hbm, x0, x1, x2, n0, n1, n2, o2d)

    o2d = _run(*xs, *ns, t, ta, tb)
    return jnp.swapaxes(
        o2d.reshape(n // 128, 4, 128)[:, :c, :], 1, 2).reshape(n, c)


def kernel(original_pos, noise, timesteps, sqrt_alphas_cumprod,
           sqrt_one_minus_alphas_cumprod):
    n, c = original_pos.shape
    ta = jnp.pad(sqrt_alphas_cumprod,
                 (0, _TABLE_PAD - sqrt_alphas_cumprod.shape[0]))
    tb = jnp.pad(sqrt_one_minus_alphas_cumprod,
                 (0, _TABLE_PAD - sqrt_one_minus_alphas_cumprod.shape[0]))

    mesh = plsc.VectorSubcoreMesh(core_axis_name="c", subcore_axis_name="s")
    cp = pltpu.CompilerParams()
    if "needs_layout_passes" in pltpu.CompilerParams.__dataclass_fields__:
        cp = dataclasses.replace(cp, needs_layout_passes=False)

    ns_slab = n // _SLABS
    outs = []
    for s in range(_SLABS):
        r0 = s * ns_slab
        xs = [original_pos[r0:r0 + ns_slab, j] for j in range(c)]
        nz = [noise[r0:r0 + ns_slab, j] for j in range(c)]
        t_s = jax.lax.slice(timesteps, (r0,), (r0 + ns_slab,))
        outs.append(_sc_slab(xs, nz, t_s, ta, tb, mesh, cp))
    if _SLABS == 1:
        return outs[0]
    return jnp.concatenate(outs, axis=0)


# speedup vs baseline: 1.2081x; 1.2081x over previous
import dataclasses
import functools

import jax
import jax.numpy as jnp
from jax.experimental import pallas as pl
from jax.experimental.pallas import tpu as pltpu
from jax.experimental.pallas import tpu_sc as plsc

_LANES = 16
_BLOCK_ROWS = 4096
_UNROLL = 4
_TABLE_PAD = 1024
_SLABS = 2


def _sc_slab(xs, ns, t, ta, tb, mesh, cp):
    n = t.shape[0]
    c = len(xs)

    @functools.partial(
        pl.kernel,
        out_type=jax.ShapeDtypeStruct((4 * n // 128, 128), jnp.float32),
        mesh=mesh,
        compiler_params=cp,
        scratch_types=[
            pltpu.VMEM((_TABLE_PAD,), jnp.float32),
            pltpu.VMEM((_TABLE_PAD,), jnp.float32),
        ],
    )
    def _run(x0, x1, x2, n0, n1, n2, t_hbm, ta_hbm, tb_hbm, o2d, ta_v, tb_v):
        pltpu.sync_copy(ta_hbm, ta_v)
        pltpu.sync_copy(tb_hbm, tb_v)

        def body(t_v, x0v, x1v, x2v, n0v, n1v, n2v, o2v):
            @plsc.parallel_loop(0, _BLOCK_ROWS, step=_LANES, unroll=_UNROLL)
            def _(k):
                sl = pl.ds(k, _LANES)
                mm = k // 128
                r = k - mm * 128
                tv = t_v[sl]
                a = plsc.load_gather(ta_v, [tv])
                b = plsc.load_gather(tb_v, [tv])
                o2v[4 * mm + 0, pl.ds(r, _LANES)] = a * x0v[sl] + b * n0v[sl]
                o2v[4 * mm + 1, pl.ds(r, _LANES)] = a * x1v[sl] + b * n1v[sl]
                o2v[4 * mm + 2, pl.ds(r, _LANES)] = a * x2v[sl] + b * n2v[sl]

        bs = pl.BlockSpec((_BLOCK_ROWS,), lambda i: (i,))
        bso = pl.BlockSpec((4 * _BLOCK_ROWS // 128, 128), lambda i: (i, 0))
        pltpu.emit_pipeline(
            body,
            grid=(n // _BLOCK_ROWS,),
            in_specs=[bs] * (2 * c + 1),
            out_specs=[bso],
            core_axis_name=("c", "s"),
            dimension_semantics=(pltpu.PARALLEL,),
        )(t_hbm, x0, x1, x2, n0, n1, n2, o2d)

    return _run(*xs, *ns, t, ta, tb)


def kernel(original_pos, noise, timesteps, sqrt_alphas_cumprod,
           sqrt_one_minus_alphas_cumprod):
    n, c = original_pos.shape
    ta = jnp.pad(sqrt_alphas_cumprod,
                 (0, _TABLE_PAD - sqrt_alphas_cumprod.shape[0]))
    tb = jnp.pad(sqrt_one_minus_alphas_cumprod,
                 (0, _TABLE_PAD - sqrt_one_minus_alphas_cumprod.shape[0]))

    mesh = plsc.VectorSubcoreMesh(core_axis_name="c", subcore_axis_name="s")
    cp = pltpu.CompilerParams()
    if "needs_layout_passes" in pltpu.CompilerParams.__dataclass_fields__:
        cp = dataclasses.replace(cp, needs_layout_passes=False)

    ns_slab = n // _SLABS
    outs = []
    for s in range(_SLABS):
        r0 = s * ns_slab
        xs = [original_pos[r0:r0 + ns_slab, j] for j in range(c)]
        nz = [noise[r0:r0 + ns_slab, j] for j in range(c)]
        t_s = jax.lax.slice(timesteps, (r0,), (r0 + ns_slab,))
        outs.append(_sc_slab(xs, nz, t_s, ta, tb, mesh, cp))
    o2d = outs[0] if _SLABS == 1 else jnp.concatenate(outs, axis=0)
    return jnp.swapaxes(
        o2d.reshape(n // 128, 4, 128)[:, :c, :], 1, 2).reshape(n, c)


# speedup vs baseline: 1.5987x; 1.3232x over previous
import dataclasses
import functools

import jax
import jax.numpy as jnp
from jax.experimental import pallas as pl
from jax.experimental.pallas import tpu as pltpu
from jax.experimental.pallas import tpu_sc as plsc

_LANES = 16
_BLOCK_ROWS = 4096
_UNROLL = 4
_TABLE_PAD = 1024
_SLABS = 1


def _sc_slab(xs, ns, t, ta, tb, mesh, cp):
    n = t.shape[0]
    c = len(xs)

    @functools.partial(
        pl.kernel,
        out_type=jax.ShapeDtypeStruct((4 * n // 128, 128), jnp.float32),
        mesh=mesh,
        compiler_params=cp,
        scratch_types=[
            pltpu.VMEM((_TABLE_PAD,), jnp.float32),
            pltpu.VMEM((_TABLE_PAD,), jnp.float32),
        ],
    )
    def _run(x0, x1, x2, n0, n1, n2, t_hbm, ta_hbm, tb_hbm, o2d, ta_v, tb_v):
        pltpu.sync_copy(ta_hbm, ta_v)
        pltpu.sync_copy(tb_hbm, tb_v)

        def body(t_v, x0v, x1v, x2v, n0v, n1v, n2v, o2v):
            @plsc.parallel_loop(0, _BLOCK_ROWS, step=_LANES, unroll=_UNROLL)
            def _(k):
                sl = pl.ds(k, _LANES)
                mm = k // 128
                r = k - mm * 128
                tv = t_v[sl]
                a = plsc.load_gather(ta_v, [tv])
                b = plsc.load_gather(tb_v, [tv])
                o2v[4 * mm + 0, pl.ds(r, _LANES)] = a * x0v[sl] + b * n0v[sl]
                o2v[4 * mm + 1, pl.ds(r, _LANES)] = a * x1v[sl] + b * n1v[sl]
                o2v[4 * mm + 2, pl.ds(r, _LANES)] = a * x2v[sl] + b * n2v[sl]

        bs = pl.BlockSpec((_BLOCK_ROWS,), lambda i: (i,))
        bso = pl.BlockSpec((4 * _BLOCK_ROWS // 128, 128), lambda i: (i, 0))
        pltpu.emit_pipeline(
            body,
            grid=(n // _BLOCK_ROWS,),
            in_specs=[bs] * (2 * c + 1),
            out_specs=[bso],
            core_axis_name=("c", "s"),
            dimension_semantics=(pltpu.PARALLEL,),
        )(t_hbm, x0, x1, x2, n0, n1, n2, o2d)

    return _run(*xs, *ns, t, ta, tb)


def kernel(original_pos, noise, timesteps, sqrt_alphas_cumprod,
           sqrt_one_minus_alphas_cumprod):
    n, c = original_pos.shape
    ta = jnp.pad(sqrt_alphas_cumprod,
                 (0, _TABLE_PAD - sqrt_alphas_cumprod.shape[0]))
    tb = jnp.pad(sqrt_one_minus_alphas_cumprod,
                 (0, _TABLE_PAD - sqrt_one_minus_alphas_cumprod.shape[0]))

    mesh = plsc.VectorSubcoreMesh(core_axis_name="c", subcore_axis_name="s")
    cp = pltpu.CompilerParams()
    if "needs_layout_passes" in pltpu.CompilerParams.__dataclass_fields__:
        cp = dataclasses.replace(cp, needs_layout_passes=False)

    ns_slab = n // _SLABS
    outs = []
    for s in range(_SLABS):
        r0 = s * ns_slab
        xs = [original_pos[r0:r0 + ns_slab, j] for j in range(c)]
        nz = [noise[r0:r0 + ns_slab, j] for j in range(c)]
        t_s = jax.lax.slice(timesteps, (r0,), (r0 + ns_slab,))
        outs.append(_sc_slab(xs, nz, t_s, ta, tb, mesh, cp))
    o2d = outs[0] if _SLABS == 1 else jnp.concatenate(outs, axis=0)
    o4 = jnp.swapaxes(o2d.reshape(n // 128, 4, 128), 1, 2).reshape(n, 4)
    return o4[:, :c]
